# baseline (device time: 32040 ns/iter reference)
import jax
import jax.numpy as jnp
from jax import lax
from jax.experimental import pallas as pl
from jax.experimental.pallas import tpu as pltpu

N_DEV = 4


def kernel(A, B):
    m, k = A.shape
    _, n = B.shape
    m_out = m // N_DEV
    nh = n // 2

    def body(a_ref, b_ref, out_ref, acc_ref,
             l1s, r1s, l1_recv, r1_recv,
             l2_stage, l2_recv, r2_stage, r2_recv,
             send_sems, recv_sems):
        my = lax.axis_index("i")
        pA = my ^ 1
        pB = 3 - my

        barrier_sem = pltpu.get_barrier_semaphore()
        for nbr in [pA, pB]:
            pl.semaphore_signal(
                barrier_sem, inc=1,
                device_id=(nbr,), device_id_type=pl.DeviceIdType.MESH,
            )
        pl.semaphore_wait(barrier_sem, 2)

        def chunk_rows(c):
            return pl.ds(c * m_out, m_out)

        def piece(c, col_off):
            return jnp.dot(
                a_ref[chunk_rows(c), :],
                b_ref[:, pl.ds(col_off, nh)],
                preferred_element_type=jnp.float32,
                precision=lax.Precision.DEFAULT,
            )

        def send_slot(src, dst, slot, sem, partner):
            r = pltpu.make_async_remote_copy(
                src_ref=src.at[slot],
                dst_ref=dst.at[slot],
                send_sem=send_sems.at[sem],
                recv_sem=recv_sems.at[sem],
                device_id=(partner,),
                device_id_type=pl.DeviceIdType.MESH,
            )
            r.start()
            return r

        pcL = my ^ 1
        pcR = 3 - my
        l1s[0] = piece(pB ^ 1, 0).astype(jnp.bfloat16)
        s0 = send_slot(l1s, l1_recv, 0, 0, pB)
        r1s[0] = piece(3 - pA, nh).astype(jnp.bfloat16)
        s1 = send_slot(r1s, r1_recv, 0, 2, pA)
        l1s[1] = piece(pB, 0).astype(jnp.bfloat16)
        s2 = send_slot(l1s, l1_recv, 1, 1, pB)
        r1s[1] = piece(pA, nh).astype(jnp.bfloat16)
        s3 = send_slot(r1s, r1_recv, 1, 3, pA)

        acc_ref[chunk_rows(pcL), pl.ds(0, nh)] = piece(pcL, 0)
        acc_ref[chunk_rows(pcR), pl.ds(nh, nh)] = piece(pcR, nh)
        out_ref[:, pl.ds(0, nh)] = piece(my, 0)
        out_ref[:, pl.ds(nh, nh)] = piece(my, nh)

        s0.wait_recv()
        l2_stage[...] = (
            acc_ref[chunk_rows(pcL), pl.ds(0, nh)]
            + l1_recv[0].astype(jnp.float32)
        ).astype(jnp.bfloat16)
        rL2 = pltpu.make_async_remote_copy(
            src_ref=l2_stage,
            dst_ref=l2_recv,
            send_sem=send_sems.at[4],
            recv_sem=recv_sems.at[4],
            device_id=(pA,),
            device_id_type=pl.DeviceIdType.MESH,
        )
        rL2.start()
        s1.wait_recv()
        r2_stage[...] = (
            acc_ref[chunk_rows(pcR), pl.ds(nh, nh)]
            + r1_recv[0].astype(jnp.float32)
        ).astype(jnp.bfloat16)
        rR2 = pltpu.make_async_remote_copy(
            src_ref=r2_stage,
            dst_ref=r2_recv,
            send_sem=send_sems.at[5],
            recv_sem=recv_sems.at[5],
            device_id=(pB,),
            device_id_type=pl.DeviceIdType.MESH,
        )
        rR2.start()

        s2.wait_recv()
        out_ref[:, pl.ds(0, nh)] += l1_recv[1].astype(jnp.float32)
        s3.wait_recv()
        out_ref[:, pl.ds(nh, nh)] += r1_recv[1].astype(jnp.float32)

        rL2.wait_recv()
        out_ref[:, pl.ds(0, nh)] += l2_recv[...].astype(jnp.float32)
        rR2.wait_recv()
        out_ref[:, pl.ds(nh, nh)] += r2_recv[...].astype(jnp.float32)

        for r in [s0, s1, s2, s3, rL2, rR2]:
            r.wait_send()

    bf = jnp.bfloat16
    return pl.pallas_call(
        body,
        out_shape=jax.ShapeDtypeStruct((m_out, n), jnp.float32),
        in_specs=[
            pl.BlockSpec(memory_space=pltpu.VMEM),
            pl.BlockSpec(memory_space=pltpu.VMEM),
        ],
        out_specs=pl.BlockSpec(memory_space=pltpu.VMEM),
        scratch_shapes=[
            pltpu.VMEM((m, n), jnp.float32),
            pltpu.VMEM((2, m_out, nh), bf),
            pltpu.VMEM((2, m_out, nh), bf),
            pltpu.VMEM((2, m_out, nh), bf),
            pltpu.VMEM((2, m_out, nh), bf),
            pltpu.VMEM((m_out, nh), bf),
            pltpu.VMEM((m_out, nh), bf),
            pltpu.VMEM((m_out, nh), bf),
            pltpu.VMEM((m_out, nh), bf),
            pltpu.SemaphoreType.DMA((6,)),
            pltpu.SemaphoreType.DMA((6,)),
        ],
        compiler_params=pltpu.CompilerParams(collective_id=0),
    )(A, B)
